# baseline (speedup 1.0000x reference)
import jax
import jax.numpy as jnp
import numpy as np
from jax.experimental import pallas as pl
from jax.experimental.pallas import tpu as pltpu
from jax.experimental.pallas import tpu_sc as plsc
import functools

R = 128
V = 100000
C_BLK = 4096
NB = (V + C_BLK - 1) // C_BLK
K = 512

_TINY = np.float32(np.finfo(np.float32).tiny)

_KS0 = np.uint32(0)
_KS1 = np.uint32(42)
_KS2 = np.uint32(np.uint32(0x1BD11BDA) ^ _KS0 ^ _KS1)

_ROT_A = (13, 15, 26, 6)
_ROT_B = (17, 29, 16, 24)

_NEG_INF = np.float32(-np.inf)
_IMAX = np.int32(2**31 - 1)


def _rotl(x, d):
    return (x << np.uint32(d)) | (x >> np.uint32(32 - d))


def _threefry_bits(n):
    x0 = jnp.zeros_like(n) + _KS0
    x1 = n + _KS1

    def rounds(x0, x1, rots):
        for r in rots:
            x0 = x0 + x1
            x1 = _rotl(x1, r)
            x1 = x0 ^ x1
        return x0, x1

    x0, x1 = rounds(x0, x1, _ROT_A)
    x0, x1 = x0 + _KS1, x1 + (_KS2 + np.uint32(1))
    x0, x1 = rounds(x0, x1, _ROT_B)
    x0, x1 = x0 + _KS2, x1 + (_KS0 + np.uint32(2))
    x0, x1 = rounds(x0, x1, _ROT_A)
    x0, x1 = x0 + _KS0, x1 + (_KS1 + np.uint32(3))
    x0, x1 = rounds(x0, x1, _ROT_B)
    x0, x1 = x0 + _KS1, x1 + (_KS2 + np.uint32(4))
    x0, x1 = rounds(x0, x1, _ROT_A)
    x0, x1 = x0 + _KS2, x1 + (_KS0 + np.uint32(5))
    return x0 ^ x1


def _gumbel_block(r_blk, c_blk, col0_u32):
    rows = jax.lax.broadcasted_iota(jnp.uint32, (r_blk, c_blk), 0)
    cols = col0_u32 + jax.lax.broadcasted_iota(jnp.uint32, (r_blk, c_blk), 1)
    n = rows * np.uint32(V) + cols
    bits = _threefry_bits(n)
    float_bits = (bits >> np.uint32(9)) | np.uint32(0x3F800000)
    u = jax.lax.bitcast_convert_type(float_bits, jnp.float32) - np.float32(1.0)
    u = jnp.maximum(_TINY, u)
    return -jnp.log(-jnp.log(u))


def _noise_kernel(out_ref):
    b = pl.program_id(0)
    col0 = (b * C_BLK).astype(jnp.uint32)
    out_ref[...] = _gumbel_block(R, C_BLK, col0)


def _make_noise():
    return pl.pallas_call(
        _noise_kernel,
        grid=(NB,),
        out_specs=pl.BlockSpec((R, C_BLK), lambda b: (0, b)),
        out_shape=jax.ShapeDtypeStruct((R, V), jnp.float32),
    )()


def _rowmax_kernel(logits_ref, out_ref, acc):
    b = pl.program_id(0)
    v = logits_ref[...]
    col0 = (b * C_BLK).astype(jnp.int32)
    cols = col0 + jax.lax.broadcasted_iota(jnp.int32, (R, C_BLK), 1)
    v = jnp.where(cols < V, v, _NEG_INF)
    m = jnp.max(v, axis=1, keepdims=True)

    @pl.when(b == 0)
    def _init():
        acc[...] = m

    @pl.when(b > 0)
    def _merge():
        acc[...] = jnp.maximum(acc[...], m)

    @pl.when(b == NB - 1)
    def _emit():
        out_ref[...] = acc[...]


def _rowmax(logits):
    return pl.pallas_call(
        _rowmax_kernel,
        grid=(NB,),
        in_specs=[pl.BlockSpec((R, C_BLK), lambda b: (0, b))],
        out_specs=pl.BlockSpec((R, 1), lambda b: (0, 0)),
        out_shape=jax.ShapeDtypeStruct((R, 1), jnp.float32),
        scratch_shapes=[pltpu.VMEM((R, 1), jnp.float32)],
    )(logits)


_NC, _NS, _L16 = 2, 16, 16
_NW = _NC * _NS
_B = R * K
_BPW = _B // _NW
V16 = (R * V) // 16


def _sc_gather_kernel(table_hbm, idx_hbm, out_hbm, idx_v, rows_v, sem):
    wid = jax.lax.axis_index("s") * _NC + jax.lax.axis_index("c")
    base = wid * _BPW
    pltpu.sync_copy(idx_hbm.at[pl.ds(base, _BPW)], idx_v)
    pltpu.async_copy(table_hbm.at[idx_v], rows_v, sem).wait()
    pltpu.sync_copy(rows_v, out_hbm.at[pl.ds(base, _BPW)])


def _sc_gather(table16, idx_flat16):
    mesh = plsc.VectorSubcoreMesh(core_axis_name="c", subcore_axis_name="s")
    fn = functools.partial(
        pl.kernel,
        mesh=mesh,
        out_type=jax.ShapeDtypeStruct((_B, 16), jnp.float32),
        scratch_types=[
            pltpu.VMEM((_BPW,), jnp.int32),
            pltpu.VMEM((_BPW, 16), jnp.float32),
            pltpu.SemaphoreType.DMA,
        ],
    )(_sc_gather_kernel)
    return fn(table16, idx_flat16)


def _cand_kernel(cand_s_ref, cand_g_ref, cand_pos_ref, gk_ref, smax_ref,
                 temp_ref, idx_ref, flag_ref):
    t = temp_ref[0, 0]
    val = cand_g_ref[...] + cand_s_ref[...] / t
    L = jnp.max(val, axis=1, keepdims=True)
    at = val == L
    idx_or_big = jnp.where(at, cand_pos_ref[...], _IMAX)
    idx_ref[...] = jnp.min(idx_or_big, axis=1, keepdims=True)
    flag = (smax_ref[...] / t + gk_ref[...]) >= L
    flag_ref[...] = flag.astype(jnp.int32)


def _cand_eval(cand_s, cand_g, cand_pos, gk, smax, temp2d):
    return pl.pallas_call(
        _cand_kernel,
        grid=(1,),
        in_specs=[
            pl.BlockSpec((R, K * 16), lambda i: (0, 0)),
            pl.BlockSpec((R, K * 16), lambda i: (0, 0)),
            pl.BlockSpec((R, K * 16), lambda i: (0, 0)),
            pl.BlockSpec((R, 1), lambda i: (0, 0)),
            pl.BlockSpec((R, 1), lambda i: (0, 0)),
            pl.BlockSpec((1, 1), lambda i: (0, 0)),
        ],
        out_specs=[
            pl.BlockSpec((R, 1), lambda i: (0, 0)),
            pl.BlockSpec((R, 1), lambda i: (0, 0)),
        ],
        out_shape=[
            jax.ShapeDtypeStruct((R, 1), jnp.int32),
            jax.ShapeDtypeStruct((R, 1), jnp.int32),
        ],
    )(cand_s, cand_g, cand_pos, gk, smax, temp2d)


def _dense_kernel(logits_ref, temp_ref, out_ref, best_val, best_idx):
    b = pl.program_id(0)
    t = temp_ref[0, 0]
    col0 = (b * C_BLK).astype(jnp.int32)
    g = _gumbel_block(R, C_BLK, col0.astype(jnp.uint32))
    val = g + logits_ref[...] / t
    cols = col0 + jax.lax.broadcasted_iota(jnp.int32, (R, C_BLK), 1)
    val = jnp.where(cols < V, val, _NEG_INF)

    local_max = jnp.max(val, axis=1, keepdims=True)
    at_max = val == local_max
    idx_or_big = jnp.where(at_max, cols, _IMAX)
    local_arg = jnp.min(idx_or_big, axis=1, keepdims=True)

    @pl.when(b == 0)
    def _init():
        best_val[...] = local_max
        best_idx[...] = local_arg

    @pl.when(b > 0)
    def _merge():
        better = local_max > best_val[...]
        best_idx[...] = jnp.where(better, local_arg, best_idx[...])
        best_val[...] = jnp.where(better, local_max, best_val[...])

    @pl.when(b == NB - 1)
    def _emit():
        out_ref[...] = best_idx[...]


def _dense_exact(logits, temp2d):
    return pl.pallas_call(
        _dense_kernel,
        grid=(NB,),
        in_specs=[
            pl.BlockSpec((R, C_BLK), lambda b: (0, b)),
            pl.BlockSpec((1, 1), lambda b: (0, 0)),
        ],
        out_specs=pl.BlockSpec((R, 1), lambda b: (0, 0)),
        out_shape=jax.ShapeDtypeStruct((R, 1), jnp.int32),
        scratch_shapes=[
            pltpu.VMEM((R, 1), jnp.float32),
            pltpu.VMEM((R, 1), jnp.int32),
        ],
        compiler_params=pltpu.CompilerParams(
            dimension_semantics=("arbitrary",),
        ),
    )(logits, temp2d)


_CACHE = None


def _candidates():
    global _CACHE
    if _CACHE is None:
        noise = jax.jit(_make_noise)()
        cg, pos = jax.lax.top_k(noise, K)
        pos = pos.astype(jnp.int32)
        flat = jnp.arange(R, dtype=jnp.int32)[:, None] * V + pos
        row16 = flat // 16
        lane = flat % 16
        oh = lane[..., None] == jnp.arange(16, dtype=jnp.int32)
        cand_g_e = jnp.where(oh, cg[..., None], _NEG_INF).reshape(R, K * 16)
        cand_pos_e = jnp.where(oh, pos[..., None], _IMAX).reshape(R, K * 16)
        _CACHE = (row16.reshape(_B), cand_g_e, cand_pos_e, cg[:, K - 1:K])
    return _CACHE


def kernel(logits, temperature):
    temp2d = temperature.reshape(1, 1)
    return _dense_exact(logits, temp2d).reshape(R)


# speedup vs baseline: 1.0997x; 1.0997x over previous
import jax
import jax.numpy as jnp
import numpy as np
from jax.experimental import pallas as pl
from jax.experimental.pallas import tpu as pltpu

R = 128
V = 100000
C_BLK = 2048
NB = (V + C_BLK - 1) // C_BLK

_TINY = np.float32(np.finfo(np.float32).tiny)

_KS0 = np.uint32(0)
_KS1 = np.uint32(42)
_KS2 = np.uint32(np.uint32(0x1BD11BDA) ^ _KS0 ^ _KS1)

_ROT_A = (13, 15, 26, 6)
_ROT_B = (17, 29, 16, 24)

_NEG_INF = np.float32(-np.inf)
_IMAX = np.int32(2**31 - 1)


def _rotl(x, d):
    return (x << np.uint32(d)) | (x >> np.uint32(32 - d))


def _threefry_bits(n):
    x0 = jnp.zeros_like(n) + _KS0
    x1 = n + _KS1

    def rounds(x0, x1, rots):
        for r in rots:
            x0 = x0 + x1
            x1 = _rotl(x1, r)
            x1 = x0 ^ x1
        return x0, x1

    x0, x1 = rounds(x0, x1, _ROT_A)
    x0, x1 = x0 + _KS1, x1 + (_KS2 + np.uint32(1))
    x0, x1 = rounds(x0, x1, _ROT_B)
    x0, x1 = x0 + _KS2, x1 + (_KS0 + np.uint32(2))
    x0, x1 = rounds(x0, x1, _ROT_A)
    x0, x1 = x0 + _KS0, x1 + (_KS1 + np.uint32(3))
    x0, x1 = rounds(x0, x1, _ROT_B)
    x0, x1 = x0 + _KS1, x1 + (_KS2 + np.uint32(4))
    x0, x1 = rounds(x0, x1, _ROT_A)
    x0, x1 = x0 + _KS2, x1 + (_KS0 + np.uint32(5))
    return x0 ^ x1


def _gumbel_block(col0_u32):
    rows = jax.lax.broadcasted_iota(jnp.uint32, (R, C_BLK), 0)
    cols = col0_u32 + jax.lax.broadcasted_iota(jnp.uint32, (R, C_BLK), 1)
    n = rows * np.uint32(V) + cols
    bits = _threefry_bits(n)
    float_bits = (bits >> np.uint32(9)) | np.uint32(0x3F800000)
    u = jax.lax.bitcast_convert_type(float_bits, jnp.float32) - np.float32(1.0)
    u = jnp.maximum(_TINY, u)
    return -jnp.log(-jnp.log(u))


def _sample_kernel(logits_ref, temp_ref, out_ref, best_val, best_idx):
    b = pl.program_id(0)
    t = temp_ref[0, 0]
    col0 = (b * C_BLK).astype(jnp.int32)
    g = _gumbel_block(col0.astype(jnp.uint32))
    val = g + logits_ref[...] / t
    cols = col0 + jax.lax.broadcasted_iota(jnp.int32, (R, C_BLK), 1)
    val = jnp.where(cols < V, val, _NEG_INF)

    local_max = jnp.max(val, axis=1, keepdims=True)
    at_max = val == local_max
    idx_or_big = jnp.where(at_max, cols, _IMAX)
    local_arg = jnp.min(idx_or_big, axis=1, keepdims=True)

    @pl.when(b == 0)
    def _init():
        best_val[...] = local_max
        best_idx[...] = local_arg

    @pl.when(b > 0)
    def _merge():
        better = local_max > best_val[...]
        best_idx[...] = jnp.where(better, local_arg, best_idx[...])
        best_val[...] = jnp.where(better, local_max, best_val[...])

    @pl.when(b == NB - 1)
    def _emit():
        out_ref[...] = best_idx[...]


@jax.jit
def _sample(logits, temp2d):
    out = pl.pallas_call(
        _sample_kernel,
        grid=(NB,),
        in_specs=[
            pl.BlockSpec((R, C_BLK), lambda b: (0, b)),
            pl.BlockSpec((1, 1), lambda b: (0, 0)),
        ],
        out_specs=pl.BlockSpec((R, 1), lambda b: (0, 0)),
        out_shape=jax.ShapeDtypeStruct((R, 1), jnp.int32),
        scratch_shapes=[
            pltpu.VMEM((R, 1), jnp.float32),
            pltpu.VMEM((R, 1), jnp.int32),
        ],
        compiler_params=pltpu.CompilerParams(
            dimension_semantics=("arbitrary",),
        ),
    )(logits, temp2d)
    return out.reshape(R)


def kernel(logits, temperature):
    return _sample(logits, temperature.reshape(1, 1))
